# baseline (speedup 1.0000x reference)
import functools

import jax
import jax.numpy as jnp
from jax import lax
from jax.experimental import pallas as pl
from jax.experimental.pallas import tpu as pltpu
from jax.experimental.pallas import tpu_sc as plsc

ATOM_FEA_LEN = 128
NUM_RADIAL = 16
OUT_LEN = 128
N_NODES = 10000
N_EDGES = 320000

_NC = 2
_NS = 16
_NW = _NC * _NS
_E_PER_W = N_EDGES // _NW
_CHUNK = 80
_N_CHUNKS = _E_PER_W // _CHUNK
_N_PAIRS = (_N_CHUNKS - 1) // 2
_SEG = 16
_BLOCKS = OUT_LEN // 32

_RBF_BLK = 3200


def _perm_cols(w):
    c = jnp.arange(OUT_LEN)
    src = (c // 32) * 32 + (c % 2) * 16 + (c % 32) // 2
    return w[:, src]


def _matmul_body(x_ref, w_ref, o_ref):
    o_ref[...] = jnp.dot(
        x_ref[...], w_ref[...],
        preferred_element_type=jnp.float32,
        precision=lax.Precision.HIGHEST,
    )


def _matmul_body_bf16_to_bf16(x_ref, w_ref, o_ref):
    o_ref[...] = jnp.dot(
        x_ref[...].astype(jnp.bfloat16), w_ref[...],
        preferred_element_type=jnp.float32,
    ).astype(jnp.bfloat16)


def _node_table(atom_fea, w01):
    return pl.pallas_call(
        _matmul_body,
        grid=(2,),
        in_specs=[
            pl.BlockSpec((N_NODES, ATOM_FEA_LEN), lambda t: (0, 0)),
            pl.BlockSpec((ATOM_FEA_LEN, OUT_LEN), lambda t: (t, 0)),
        ],
        out_specs=pl.BlockSpec((N_NODES, OUT_LEN), lambda t: (t, 0)),
        out_shape=jax.ShapeDtypeStruct((2 * N_NODES, OUT_LEN), jnp.float32),
    )(atom_fea, w01)


def _rbf_proj(rbf, w2_bf16):
    return pl.pallas_call(
        _matmul_body_bf16_to_bf16,
        grid=(N_EDGES // _RBF_BLK,),
        in_specs=[
            pl.BlockSpec((_RBF_BLK, NUM_RADIAL), lambda t: (t, 0)),
            pl.BlockSpec((NUM_RADIAL, OUT_LEN), lambda t: (0, 0)),
        ],
        out_specs=pl.BlockSpec((_RBF_BLK, OUT_LEN), lambda t: (t, 0)),
        out_shape=jax.ShapeDtypeStruct((N_EDGES, OUT_LEN), jnp.bfloat16),
    )(rbf, w2_bf16)


def _sc_body(t_hbm, i0_hbm, i1_hbm, r_hbm, out_hbm,
             i0_v, i1_v, g0_v, g1_v, r_v, sem0, sem1):
    sems = (sem0, sem1)
    wid = lax.axis_index("s") * _NC + lax.axis_index("c")
    base = wid * _E_PER_W

    pltpu.sync_copy(i0_hbm.at[pl.ds(base, _E_PER_W)], i0_v)
    pltpu.sync_copy(i1_hbm.at[pl.ds(base, _E_PER_W)], i1_v)

    def issue(b, ci):
        off = ci * _CHUNK
        pltpu.async_copy(t_hbm.at[i0_v.at[pl.ds(off, _CHUNK)]], g0_v.at[b],
                         sems[b])
        pltpu.async_copy(t_hbm.at[i1_v.at[pl.ds(off, _CHUNK)]], g1_v.at[b],
                         sems[b])
        pltpu.async_copy(r_hbm.at[pl.ds(base + off, _CHUNK)], r_v.at[b],
                         sems[b])

    def drain(b):
        dummy32 = t_hbm.at[pl.ds(0, _CHUNK)]
        dummyw = r_hbm.at[pl.ds(0, _CHUNK)]
        pltpu.make_async_copy(dummy32, g0_v.at[b], sems[b]).wait()
        pltpu.make_async_copy(dummy32, g1_v.at[b], sems[b]).wait()
        pltpu.make_async_copy(dummyw, r_v.at[b], sems[b]).wait()

    def combine_and_store(b, ci):
        himask = jnp.int32(-65536)
        bc = lambda v: lax.bitcast_convert_type(v, jnp.float32)

        def row_body(r, carry):
            for c in range(_BLOCKS):
                sla = pl.ds(c * 32, _SEG)
                slb = pl.ds(c * 32 + 16, _SEG)
                wr = r_v[b, r, pl.ds(c * 16, 16)]
                g0_v[b, r, sla] = (g0_v[b, r, sla] + g1_v[b, r, sla]
                                   + bc(wr << 16))
                g0_v[b, r, slb] = (g0_v[b, r, slb] + g1_v[b, r, slb]
                                   + bc(wr & himask))
            return carry

        lax.fori_loop(0, _CHUNK, row_body, 0)
        pltpu.sync_copy(g0_v.at[b],
                        out_hbm.at[pl.ds(base + ci * _CHUNK, _CHUNK)])

    issue(0, 0)

    def pair_body(p, carry):
        issue(1, 2 * p + 1)
        drain(0)
        combine_and_store(0, 2 * p)
        issue(0, 2 * p + 2)
        drain(1)
        combine_and_store(1, 2 * p + 1)
        return carry

    lax.fori_loop(0, _N_PAIRS, pair_body, 0)
    drain(0)
    combine_and_store(0, _N_CHUNKS - 1)


@functools.partial(
    pl.kernel,
    out_type=jax.ShapeDtypeStruct((N_EDGES, OUT_LEN), jnp.float32),
    mesh=plsc.VectorSubcoreMesh(core_axis_name="c", subcore_axis_name="s"),
    scratch_types=[
        pltpu.VMEM((_E_PER_W,), jnp.int32),
        pltpu.VMEM((_E_PER_W,), jnp.int32),
        pltpu.VMEM((2, _CHUNK, OUT_LEN), jnp.float32),
        pltpu.VMEM((2, _CHUNK, OUT_LEN), jnp.float32),
        pltpu.VMEM((2, _CHUNK, OUT_LEN // 2), jnp.int32),
        pltpu.SemaphoreType.DMA,
        pltpu.SemaphoreType.DMA,
    ],
)
def _sc_combine(t_hbm, i0_hbm, i1_hbm, r_hbm, out_hbm, *scratch):
    _sc_body(t_hbm, i0_hbm, i1_hbm, r_hbm, out_hbm, *scratch)


def _as_i32_words(x_bf16):
    n = x_bf16.shape[0]
    return lax.bitcast_convert_type(
        x_bf16.reshape(n, OUT_LEN // 2, 2), jnp.int32)


def kernel(atom_fea, rbf, nbr_fea_idx, W):
    w01 = W[: 2 * ATOM_FEA_LEN]
    w2_bf16 = _perm_cols(W[2 * ATOM_FEA_LEN :]).astype(jnp.bfloat16)
    table = _node_table(atom_fea, w01)
    r = _as_i32_words(_rbf_proj(rbf, w2_bf16))
    i0 = nbr_fea_idx[:, 0]
    i1 = nbr_fea_idx[:, 1] + N_NODES
    return _sc_combine(table, i0, i1, r)


# speedup vs baseline: 2.5436x; 2.5436x over previous
import functools

import jax
import jax.numpy as jnp
from jax import lax
from jax.experimental import pallas as pl
from jax.experimental.pallas import tpu as pltpu
from jax.experimental.pallas import tpu_sc as plsc

ATOM_FEA_LEN = 128
NUM_RADIAL = 16
OUT_LEN = 128
N_NODES = 10000
N_EDGES = 320000

_NC = 2
_NS = 16
_NW = _NC * _NS
_E_PER_W = N_EDGES // _NW
_CHUNK = 80
_N_CHUNKS = _E_PER_W // _CHUNK
_N_PAIRS = (_N_CHUNKS - 1) // 2
_SEG = 16
_BLOCKS = OUT_LEN // 32

_RBF_BLK = 3200


def _matmul_body(x_ref, w_ref, o_ref):
    o_ref[...] = jnp.dot(
        x_ref[...], w_ref[...],
        preferred_element_type=jnp.float32,
        precision=lax.Precision.HIGHEST,
    )


def _matmul_body_bf16_packed(x_ref, w_ref, o_ref):
    y = jnp.dot(
        x_ref[...].astype(jnp.bfloat16), w_ref[...],
        preferred_element_type=jnp.float32,
    )
    half = jnp.int32(0x8000)
    a = lax.bitcast_convert_type(y[:, : OUT_LEN // 2], jnp.int32) + half
    b = lax.bitcast_convert_type(y[:, OUT_LEN // 2 :], jnp.int32) + half
    o_ref[...] = ((b & jnp.int32(-65536))
                  | ((a >> 16) & jnp.int32(0xFFFF)))


def _node_table(atom_fea, w01):
    return pl.pallas_call(
        _matmul_body,
        grid=(2,),
        in_specs=[
            pl.BlockSpec((N_NODES, ATOM_FEA_LEN), lambda t: (0, 0)),
            pl.BlockSpec((ATOM_FEA_LEN, OUT_LEN), lambda t: (t, 0)),
        ],
        out_specs=pl.BlockSpec((N_NODES, OUT_LEN), lambda t: (t, 0)),
        out_shape=jax.ShapeDtypeStruct((2 * N_NODES, OUT_LEN), jnp.float32),
    )(atom_fea, w01)


def _rbf_proj(rbf, w2_bf16):
    return pl.pallas_call(
        _matmul_body_bf16_packed,
        grid=(N_EDGES // _RBF_BLK,),
        in_specs=[
            pl.BlockSpec((_RBF_BLK, NUM_RADIAL), lambda t: (t, 0)),
            pl.BlockSpec((NUM_RADIAL, OUT_LEN), lambda t: (0, 0)),
        ],
        out_specs=pl.BlockSpec((_RBF_BLK, OUT_LEN // 2), lambda t: (t, 0)),
        out_shape=jax.ShapeDtypeStruct((N_EDGES, OUT_LEN // 2), jnp.int32),
    )(rbf, w2_bf16)


def _sc_body(t_hbm, i0_hbm, i1_hbm, r_hbm, out_hbm,
             i0_v, i1_v, g0_v, g1_v, r_v, sem0, sem1):
    sems = (sem0, sem1)
    wid = lax.axis_index("s") * _NC + lax.axis_index("c")
    base = wid * _E_PER_W

    pltpu.sync_copy(i0_hbm.at[pl.ds(base, _E_PER_W)], i0_v)
    pltpu.sync_copy(i1_hbm.at[pl.ds(base, _E_PER_W)], i1_v)

    def issue(b, ci):
        off = ci * _CHUNK
        pltpu.async_copy(t_hbm.at[i0_v.at[pl.ds(off, _CHUNK)]], g0_v.at[b],
                         sems[b])
        pltpu.async_copy(t_hbm.at[i1_v.at[pl.ds(off, _CHUNK)]], g1_v.at[b],
                         sems[b])
        pltpu.async_copy(r_hbm.at[pl.ds(base + off, _CHUNK)], r_v.at[b],
                         sems[b])

    def drain(b):
        dummy32 = t_hbm.at[pl.ds(0, _CHUNK)]
        dummyw = r_hbm.at[pl.ds(0, _CHUNK)]
        pltpu.make_async_copy(dummy32, g0_v.at[b], sems[b]).wait()
        pltpu.make_async_copy(dummy32, g1_v.at[b], sems[b]).wait()
        pltpu.make_async_copy(dummyw, r_v.at[b], sems[b]).wait()

    def combine_and_store(b, ci):
        himask = jnp.int32(-65536)
        bc = lambda v: lax.bitcast_convert_type(v, jnp.float32)

        def row_body(r, carry):
            for c in range(_BLOCKS):
                sla = pl.ds(c * _SEG, _SEG)
                slb = pl.ds(OUT_LEN // 2 + c * _SEG, _SEG)
                wr = r_v[b, r, pl.ds(c * _SEG, _SEG)]
                g0_v[b, r, sla] = (g0_v[b, r, sla] + g1_v[b, r, sla]
                                   + bc(wr << 16))
                g0_v[b, r, slb] = (g0_v[b, r, slb] + g1_v[b, r, slb]
                                   + bc(wr & himask))
            return carry

        lax.fori_loop(0, _CHUNK, row_body, 0)
        pltpu.sync_copy(g0_v.at[b],
                        out_hbm.at[pl.ds(base + ci * _CHUNK, _CHUNK)])

    issue(0, 0)

    def pair_body(p, carry):
        issue(1, 2 * p + 1)
        drain(0)
        combine_and_store(0, 2 * p)
        issue(0, 2 * p + 2)
        drain(1)
        combine_and_store(1, 2 * p + 1)
        return carry

    lax.fori_loop(0, _N_PAIRS, pair_body, 0)
    drain(0)
    combine_and_store(0, _N_CHUNKS - 1)


@functools.partial(
    pl.kernel,
    out_type=jax.ShapeDtypeStruct((N_EDGES, OUT_LEN), jnp.float32),
    mesh=plsc.VectorSubcoreMesh(core_axis_name="c", subcore_axis_name="s"),
    scratch_types=[
        pltpu.VMEM((_E_PER_W,), jnp.int32),
        pltpu.VMEM((_E_PER_W,), jnp.int32),
        pltpu.VMEM((2, _CHUNK, OUT_LEN), jnp.float32),
        pltpu.VMEM((2, _CHUNK, OUT_LEN), jnp.float32),
        pltpu.VMEM((2, _CHUNK, OUT_LEN // 2), jnp.int32),
        pltpu.SemaphoreType.DMA,
        pltpu.SemaphoreType.DMA,
    ],
)
def _sc_combine(t_hbm, i0_hbm, i1_hbm, r_hbm, out_hbm, *scratch):
    _sc_body(t_hbm, i0_hbm, i1_hbm, r_hbm, out_hbm, *scratch)


def kernel(atom_fea, rbf, nbr_fea_idx, W):
    w01 = W[: 2 * ATOM_FEA_LEN]
    w2_bf16 = W[2 * ATOM_FEA_LEN :].astype(jnp.bfloat16)
    table = _node_table(atom_fea, w01)
    r = _rbf_proj(rbf, w2_bf16)
    i0 = nbr_fea_idx[:, 0]
    i1 = nbr_fea_idx[:, 1] + N_NODES
    return _sc_combine(table, i0, i1, r)


# speedup vs baseline: 2.6645x; 1.0475x over previous
import functools

import jax
import jax.numpy as jnp
from jax import lax
from jax.experimental import pallas as pl
from jax.experimental.pallas import tpu as pltpu
from jax.experimental.pallas import tpu_sc as plsc

ATOM_FEA_LEN = 128
NUM_RADIAL = 16
OUT_LEN = 128
N_NODES = 10000
N_EDGES = 320000

_NC = 2
_NS = 16
_NW = _NC * _NS
_E_PER_W = N_EDGES // _NW
_CHUNK = 80
_N_CHUNKS = _E_PER_W // _CHUNK
_N_PAIRS = (_N_CHUNKS - 1) // 2
_SEG = 16
_NSEG = OUT_LEN // _SEG

_RBF_BLK = 3200


def _matmul_body(x_ref, w_ref, o_ref):
    o_ref[...] = jnp.dot(
        x_ref[...], w_ref[...],
        preferred_element_type=jnp.float32,
        precision=lax.Precision.HIGHEST,
    )


def _matmul_body_bf16(x_ref, w_ref, o_ref):
    o_ref[...] = jnp.dot(
        x_ref[...].astype(jnp.bfloat16), w_ref[...],
        preferred_element_type=jnp.float32,
    )


def _node_table(atom_fea, w01):
    return pl.pallas_call(
        _matmul_body,
        grid=(2,),
        in_specs=[
            pl.BlockSpec((N_NODES, ATOM_FEA_LEN), lambda t: (0, 0)),
            pl.BlockSpec((ATOM_FEA_LEN, OUT_LEN), lambda t: (t, 0)),
        ],
        out_specs=pl.BlockSpec((N_NODES, OUT_LEN), lambda t: (t, 0)),
        out_shape=jax.ShapeDtypeStruct((2 * N_NODES, OUT_LEN), jnp.float32),
    )(atom_fea, w01)


def _rbf_proj(rbf, w2_bf16):
    return pl.pallas_call(
        _matmul_body_bf16,
        grid=(N_EDGES // _RBF_BLK,),
        in_specs=[
            pl.BlockSpec((_RBF_BLK, NUM_RADIAL), lambda t: (t, 0)),
            pl.BlockSpec((NUM_RADIAL, OUT_LEN), lambda t: (0, 0)),
        ],
        out_specs=pl.BlockSpec((_RBF_BLK, OUT_LEN), lambda t: (t, 0)),
        out_shape=jax.ShapeDtypeStruct((N_EDGES, OUT_LEN), jnp.float32),
    )(rbf, w2_bf16)


def _sc_body(t_hbm, i0_hbm, i1_hbm, r_hbm, out_hbm,
             i0_v, i1_v, g0_v, g1_v, r_v, sem0, sem1):
    sems = (sem0, sem1)
    wid = lax.axis_index("s") * _NC + lax.axis_index("c")
    base = wid * _E_PER_W

    pltpu.sync_copy(i0_hbm.at[pl.ds(base, _E_PER_W)], i0_v)
    pltpu.sync_copy(i1_hbm.at[pl.ds(base, _E_PER_W)], i1_v)

    def issue(b, ci):
        off = ci * _CHUNK
        pltpu.async_copy(t_hbm.at[i0_v.at[pl.ds(off, _CHUNK)]], g0_v.at[b],
                         sems[b])
        pltpu.async_copy(t_hbm.at[i1_v.at[pl.ds(off, _CHUNK)]], g1_v.at[b],
                         sems[b])
        pltpu.async_copy(r_hbm.at[pl.ds(base + off, _CHUNK)], r_v.at[b],
                         sems[b])

    def drain(b):
        dummy = t_hbm.at[pl.ds(0, _CHUNK)]
        pltpu.make_async_copy(dummy, g0_v.at[b], sems[b]).wait()
        pltpu.make_async_copy(dummy, g1_v.at[b], sems[b]).wait()
        pltpu.make_async_copy(dummy, r_v.at[b], sems[b]).wait()

    def combine_and_store(b, ci):
        def row_body(r, carry):
            for c in range(_NSEG):
                seg = pl.ds(c * _SEG, _SEG)
                g0_v[b, r, seg] = (g0_v[b, r, seg] + g1_v[b, r, seg]
                                   + r_v[b, r, seg])
            return carry

        lax.fori_loop(0, _CHUNK, row_body, 0)
        pltpu.sync_copy(g0_v.at[b],
                        out_hbm.at[pl.ds(base + ci * _CHUNK, _CHUNK)])

    issue(0, 0)

    def pair_body(p, carry):
        issue(1, 2 * p + 1)
        drain(0)
        combine_and_store(0, 2 * p)
        issue(0, 2 * p + 2)
        drain(1)
        combine_and_store(1, 2 * p + 1)
        return carry

    lax.fori_loop(0, _N_PAIRS, pair_body, 0)
    drain(0)
    combine_and_store(0, _N_CHUNKS - 1)


@functools.partial(
    pl.kernel,
    out_type=jax.ShapeDtypeStruct((N_EDGES, OUT_LEN), jnp.float32),
    mesh=plsc.VectorSubcoreMesh(core_axis_name="c", subcore_axis_name="s"),
    scratch_types=[
        pltpu.VMEM((_E_PER_W,), jnp.int32),
        pltpu.VMEM((_E_PER_W,), jnp.int32),
        pltpu.VMEM((2, _CHUNK, OUT_LEN), jnp.float32),
        pltpu.VMEM((2, _CHUNK, OUT_LEN), jnp.float32),
        pltpu.VMEM((2, _CHUNK, OUT_LEN), jnp.float32),
        pltpu.SemaphoreType.DMA,
        pltpu.SemaphoreType.DMA,
    ],
)
def _sc_combine(t_hbm, i0_hbm, i1_hbm, r_hbm, out_hbm, *scratch):
    _sc_body(t_hbm, i0_hbm, i1_hbm, r_hbm, out_hbm, *scratch)


def kernel(atom_fea, rbf, nbr_fea_idx, W):
    w01 = W[: 2 * ATOM_FEA_LEN]
    w2_bf16 = W[2 * ATOM_FEA_LEN :].astype(jnp.bfloat16)
    table = _node_table(atom_fea, w01)
    r = _rbf_proj(rbf, w2_bf16)
    i0 = nbr_fea_idx[:, 0]
    i1 = nbr_fea_idx[:, 1] + N_NODES
    return _sc_combine(table, i0, i1, r)
